# Initial kernel scaffold; baseline (speedup 1.0000x reference)
import jax
import jax.numpy as jnp
from jax.experimental import pallas as pl


def kernel(x, adj1, y, adj2, W1, b1, W2, b2, W3, b3, W4, b4, W5, b5, Wm, bm):
    raise NotImplementedError("write your pallas kernel here")


# speedup vs baseline: 1.7299x; 1.7299x over previous
import jax
import jax.numpy as jnp
from jax.experimental import pallas as pl
from jax.experimental.pallas import tpu as pltpu

_N = 4096
_NFEAT = 128
_F0, _F1, _F2 = 64, 68, 72
_SUMF = _F0 + _F1 + _F2
_H4 = _F0 * 2 + 4
_H5 = _F0 * 2
_BM = 256
_NB = _N // _BM


def _dot(a, b):
    return jnp.dot(a, b, preferred_element_type=jnp.float32)


def _phase1_body(adj_ref, x_ref, wl_ref, bl_ref, lr_ref):
    ax = _dot(adj_ref[...], x_ref[...])
    s = jax.nn.sigmoid(_dot(ax, wl_ref[...]) + bl_ref[...])
    fir = s[:, :_F0]
    sec = s[:, _F0:_F0 + _F1]
    thi = s[:, _F0 + _F1:]
    f2 = jnp.mean(sec, axis=1, keepdims=True) * thi
    lr_ref[...] = jnp.concatenate([fir, sec, f2], axis=1)


def _phase2_body(adj_ref, lr_full_ref, lr_blk_ref, w4_ref, b4_ref,
                 w5_ref, b5_ref, wmt_ref, bm_ref,
                 final_ref, fiv_ref, mlp_ref):
    alr = _dot(adj_ref[...], lr_full_ref[...])
    fou = _dot(alr, w4_ref[...]) + b4_ref[...]
    fiv = _dot(alr, w5_ref[...]) + b5_ref[...]
    m = _dot(fiv, wmt_ref[...]) + bm_ref[...]
    m = jnp.where(m >= 0, m, 0.01 * m)
    f3 = (m + fou) * 0.5
    lrb = lr_blk_ref[...]
    low = jnp.mean(lrb, axis=1, keepdims=True) * lrb + lrb
    final_ref[...] = jnp.concatenate([low, f3], axis=1)
    fiv_ref[...] = fiv
    mlp_ref[...] = m


def _branch(x, adj, wl, bl, w4, b4, w5, b5, wmt, bm2):
    f32 = jnp.float32
    lr = pl.pallas_call(
        _phase1_body,
        grid=(_NB,),
        in_specs=[
            pl.BlockSpec((_BM, _N), lambda i: (i, 0)),
            pl.BlockSpec((_N, _NFEAT), lambda i: (0, 0)),
            pl.BlockSpec((_NFEAT, _SUMF), lambda i: (0, 0)),
            pl.BlockSpec((1, _SUMF), lambda i: (0, 0)),
        ],
        out_specs=pl.BlockSpec((_BM, _SUMF), lambda i: (i, 0)),
        out_shape=jax.ShapeDtypeStruct((_N, _SUMF), f32),
        compiler_params=pltpu.CompilerParams(
            dimension_semantics=("arbitrary",)),
    )(adj, x, wl, bl)

    final, fiv, mlp = pl.pallas_call(
        _phase2_body,
        grid=(_NB,),
        in_specs=[
            pl.BlockSpec((_BM, _N), lambda i: (i, 0)),
            pl.BlockSpec((_N, _SUMF), lambda i: (0, 0)),
            pl.BlockSpec((_BM, _SUMF), lambda i: (i, 0)),
            pl.BlockSpec((_SUMF, _H4), lambda i: (0, 0)),
            pl.BlockSpec((1, _H4), lambda i: (0, 0)),
            pl.BlockSpec((_SUMF, _H5), lambda i: (0, 0)),
            pl.BlockSpec((1, _H5), lambda i: (0, 0)),
            pl.BlockSpec((_H5, _H4), lambda i: (0, 0)),
            pl.BlockSpec((1, _H4), lambda i: (0, 0)),
        ],
        out_specs=[
            pl.BlockSpec((_BM, _SUMF + _H4), lambda i: (i, 0)),
            pl.BlockSpec((_BM, _H5), lambda i: (i, 0)),
            pl.BlockSpec((_BM, _H4), lambda i: (i, 0)),
        ],
        out_shape=[
            jax.ShapeDtypeStruct((_N, _SUMF + _H4), f32),
            jax.ShapeDtypeStruct((_N, _H5), f32),
            jax.ShapeDtypeStruct((_N, _H4), f32),
        ],
        compiler_params=pltpu.CompilerParams(
            dimension_semantics=("arbitrary",)),
    )(adj, lr, lr, w4, b4, w5, b5, wmt, bm2)
    return lr, final, fiv, mlp


def kernel(x, adj1, y, adj2, W1, b1, W2, b2, W3, b3, W4, b4, W5, b5, Wm, bm):
    wl = jnp.concatenate([W1, W2, W3], axis=1)
    bl = jnp.concatenate([b1, b2, b3]).reshape(1, _SUMF)
    b4r = b4.reshape(1, _H4)
    b5r = b5.reshape(1, _H5)
    wmt = Wm.T
    bmr = bm.reshape(1, _H4)
    x_lr, x_final, x_fiv, x_mlp = _branch(
        x, adj1, wl, bl, W4, b4r, W5, b5r, wmt, bmr)
    y_lr, y_final, y_fiv, y_mlp = _branch(
        y, adj2, wl, bl, W4, b4r, W5, b5r, wmt, bmr)
    return (x_lr, y_lr, x_final, y_final, x_fiv, x_mlp, y_fiv, y_mlp)


# speedup vs baseline: 1.8800x; 1.0867x over previous
import jax
import jax.numpy as jnp
from jax.experimental import pallas as pl
from jax.experimental.pallas import tpu as pltpu

_N = 4096
_NFEAT = 128
_F0, _F1, _F2 = 64, 68, 72
_SUMF = _F0 + _F1 + _F2
_H4 = _F0 * 2 + 4
_H5 = _F0 * 2
_BM = 512
_NB = _N // _BM


def _dot(a, b):
    return jnp.dot(a, b, preferred_element_type=jnp.float32)


def _phase1_body(adj_ref, x_ref, wl_ref, bl_ref, lr_ref):
    ax = _dot(adj_ref[...], x_ref[...])
    s = jax.nn.sigmoid(_dot(ax, wl_ref[...]) + bl_ref[...])
    fir = s[:, :_F0]
    sec = s[:, _F0:_F0 + _F1]
    thi = s[:, _F0 + _F1:]
    f2 = jnp.mean(sec, axis=1, keepdims=True) * thi
    lr_ref[...] = jnp.concatenate([fir, sec, f2], axis=1)


def _phase2_body(adj_ref, lr_full_ref, lr_blk_ref, w4_ref, b4_ref,
                 w5_ref, b5_ref, wmt_ref, bm_ref,
                 final_ref, fiv_ref, mlp_ref):
    alr = _dot(adj_ref[...], lr_full_ref[...])
    fou = _dot(alr, w4_ref[...]) + b4_ref[...]
    fiv = _dot(alr, w5_ref[...]) + b5_ref[...]
    m = _dot(fiv, wmt_ref[...]) + bm_ref[...]
    m = jnp.where(m >= 0, m, 0.01 * m)
    f3 = (m + fou) * 0.5
    lrb = lr_blk_ref[...]
    low = jnp.mean(lrb, axis=1, keepdims=True) * lrb + lrb
    final_ref[...] = jnp.concatenate([low, f3], axis=1)
    fiv_ref[...] = fiv
    mlp_ref[...] = m


def _branch(x, adj, wl, bl, w4, b4, w5, b5, wmt, bm2):
    f32 = jnp.float32
    lr = pl.pallas_call(
        _phase1_body,
        grid=(_NB,),
        in_specs=[
            pl.BlockSpec((_BM, _N), lambda i: (i, 0)),
            pl.BlockSpec((_N, _NFEAT), lambda i: (0, 0)),
            pl.BlockSpec((_NFEAT, _SUMF), lambda i: (0, 0)),
            pl.BlockSpec((1, _SUMF), lambda i: (0, 0)),
        ],
        out_specs=pl.BlockSpec((_BM, _SUMF), lambda i: (i, 0)),
        out_shape=jax.ShapeDtypeStruct((_N, _SUMF), f32),
        compiler_params=pltpu.CompilerParams(
            dimension_semantics=("arbitrary",)),
    )(adj, x, wl, bl)

    final, fiv, mlp = pl.pallas_call(
        _phase2_body,
        grid=(_NB,),
        in_specs=[
            pl.BlockSpec((_BM, _N), lambda i: (i, 0)),
            pl.BlockSpec((_N, _SUMF), lambda i: (0, 0)),
            pl.BlockSpec((_BM, _SUMF), lambda i: (i, 0)),
            pl.BlockSpec((_SUMF, _H4), lambda i: (0, 0)),
            pl.BlockSpec((1, _H4), lambda i: (0, 0)),
            pl.BlockSpec((_SUMF, _H5), lambda i: (0, 0)),
            pl.BlockSpec((1, _H5), lambda i: (0, 0)),
            pl.BlockSpec((_H5, _H4), lambda i: (0, 0)),
            pl.BlockSpec((1, _H4), lambda i: (0, 0)),
        ],
        out_specs=[
            pl.BlockSpec((_BM, _SUMF + _H4), lambda i: (i, 0)),
            pl.BlockSpec((_BM, _H5), lambda i: (i, 0)),
            pl.BlockSpec((_BM, _H4), lambda i: (i, 0)),
        ],
        out_shape=[
            jax.ShapeDtypeStruct((_N, _SUMF + _H4), f32),
            jax.ShapeDtypeStruct((_N, _H5), f32),
            jax.ShapeDtypeStruct((_N, _H4), f32),
        ],
        compiler_params=pltpu.CompilerParams(
            dimension_semantics=("arbitrary",)),
    )(adj, lr, lr, w4, b4, w5, b5, wmt, bm2)
    return lr, final, fiv, mlp


def kernel(x, adj1, y, adj2, W1, b1, W2, b2, W3, b3, W4, b4, W5, b5, Wm, bm):
    wl = jnp.concatenate([W1, W2, W3], axis=1)
    bl = jnp.concatenate([b1, b2, b3]).reshape(1, _SUMF)
    b4r = b4.reshape(1, _H4)
    b5r = b5.reshape(1, _H5)
    wmt = Wm.T
    bmr = bm.reshape(1, _H4)
    x_lr, x_final, x_fiv, x_mlp = _branch(
        x, adj1, wl, bl, W4, b4r, W5, b5r, wmt, bmr)
    y_lr, y_final, y_fiv, y_mlp = _branch(
        y, adj2, wl, bl, W4, b4r, W5, b5r, wmt, bmr)
    return (x_lr, y_lr, x_final, y_final, x_fiv, x_mlp, y_fiv, y_mlp)
